# baseline (speedup 1.0000x reference)
import functools
import numpy as np
import jax
import jax.numpy as jnp
from jax import lax
from jax.experimental import pallas as pl
from jax.experimental.pallas import tpu as pltpu
from jax.experimental.pallas import tpu_sc as plsc

_B, _N, _K, _CP = 4, 4096, 20, 16
_RBLK = 256
_P = 256
_SQ = float(np.sqrt(np.float32(1.0 + 1e-5)))

_NC, _NS = 2, 16
_NW = _NC * _NS
_TOT = _B * _K * _N
_PW = _TOT // _NW
_CHUNK = 128
_NCHUNK = _PW // _CHUNK
_GCH = 16
_GR = _GCH * _CHUNK
_NGRP = _NCHUNK // _GCH


def _knn_body(xt_ref, xa_ref, idx_ref):
    b = pl.program_id(0)
    xt = xt_ref[0]
    xa = xa_ref[0]
    g = jnp.dot(xt, xa, preferred_element_type=jnp.float32)
    sqr = xt * xt
    sqc = xa * xa
    rn = sqr[:, 0:1]
    cn = sqc[0:1, :]
    for c in range(1, 9):
        rn = rn + sqr[:, c:c + 1]
        cn = cn + sqc[c:c + 1, :]
    d = (2.0 * g - cn) - rn
    fiota = -lax.broadcasted_iota(jnp.int32, d.shape, 1).astype(jnp.float32)
    neg = jnp.float32(-jnp.inf)
    negbig = jnp.float32(-3.0e38)
    cols = []
    for _ in range(_K):
        m = jnp.max(d, axis=1, keepdims=True)
        sel = jnp.where(d == m, fiota, negbig)
        mx = jnp.max(sel, axis=1, keepdims=True)
        cols.append(mx)
        d = jnp.where(sel == mx, neg, d)
    idxf = jnp.concatenate(cols, axis=1)
    idx_ref[0] = b * _N - idxf.T.astype(jnp.int32)


def _sc_gather_body(table_ref, idx_ref, out_ref, idx_v, rows_v, sem):
    wid = lax.axis_index("s") * _NC + lax.axis_index("c")
    pltpu.sync_copy(idx_ref.at[wid], idx_v)

    def group(gi, carry):
        cps = [
            pltpu.async_copy(
                table_ref.at[idx_v.at[gi * _GCH + j]],
                rows_v.at[pl.ds(j * _CHUNK, _CHUNK)],
                sem,
            )
            for j in range(_GCH)
        ]
        for cp in cps:
            cp.wait()
        pltpu.sync_copy(rows_v, out_ref.at[pl.ds(wid * _PW + gi * _GR, _GR)])
        return carry

    lax.fori_loop(0, _NGRP, group, 0)


def _sc_gather(table, idx3):
    mesh = plsc.VectorSubcoreMesh(core_axis_name="c", subcore_axis_name="s")
    f = pl.kernel(
        _sc_gather_body,
        mesh=mesh,
        out_type=jax.ShapeDtypeStruct((_TOT, _CP), jnp.float32),
        scratch_types=[
            pltpu.VMEM((_NCHUNK, _CHUNK), jnp.int32),
            pltpu.VMEM((_GR, _CP), jnp.float32),
            pltpu.SemaphoreType.DMA,
        ],
        compiler_params=pltpu.CompilerParams(use_tc_tiling_on_sc=False),
    )
    return f(table, idx3)


def _lrelu(h):
    return jnp.where(h >= 0, h, 0.2 * h)


def _bn(h, g, b):
    return h * (g[...] / _SQ) + b[...]


def _k3_body(feat_ref, xe_ref,
             w1t, w11, w2, g1, b1, g11, b11, g2, b2,
             w3t, w31, w4, g3, b3, g31, b31, g4, b4,
             w5t, w51, w52, g5, b5, g51, b51, g52, b52,
             w6, g6, b6,
             x123_ref, gmax_ref):
    nb = pl.program_id(1)
    f20 = feat_ref[0].reshape(_K * _P, _CP)
    xe = xe_ref[0]

    def branch(k, wa, wb, wc, ga, ba, gb, bb, gc, bc):
        f = f20[: k * _P]
        xe_t = jnp.concatenate([xe] * k, axis=0)
        e1in = jnp.concatenate([f - xe_t, xe_t], axis=1)
        e1 = _lrelu(_bn(jnp.dot(e1in, wa[...], preferred_element_type=jnp.float32), ga, ba))
        e2 = _lrelu(_bn(jnp.dot(e1, wb[...], preferred_element_type=jnp.float32), gb, bb))
        e3 = _lrelu(_bn(jnp.dot(e2, wc[...], preferred_element_type=jnp.float32), gc, bc))
        m = e3[:_P]
        for j in range(1, k):
            m = jnp.maximum(m, e3[j * _P:(j + 1) * _P])
        return m

    x1 = branch(20, w1t, w11, w2, g1, b1, g11, b11, g2, b2)
    x2 = branch(6, w3t, w31, w4, g3, b3, g31, b31, g4, b4)
    x3 = branch(8, w5t, w51, w52, g5, b5, g51, b51, g52, b52)
    x123 = jnp.concatenate([x1, x2, x3], axis=1)
    x123_ref[0] = x123
    h6 = _lrelu(_bn(jnp.dot(x123, w6[...], preferred_element_type=jnp.float32), g6, b6))
    bm = jnp.max(h6, axis=0, keepdims=True)

    @pl.when(nb == 0)
    def _init():
        gmax_ref[0] = bm

    @pl.when(nb > 0)
    def _acc():
        gmax_ref[0] = jnp.maximum(gmax_ref[0], bm)


def _k4_body(x123_ref, g_ref, w7g, w7x, g7, b7, w8, g8, b8, w9, out_ref):
    x123 = x123_ref[0]
    gv = g_ref[0]
    c = jnp.dot(gv, w7g[...], preferred_element_type=jnp.float32)
    h7 = _lrelu(_bn(jnp.dot(x123, w7x[...], preferred_element_type=jnp.float32) + c, g7, b7))
    h8 = _lrelu(_bn(jnp.dot(h7, w8[...], preferred_element_type=jnp.float32), g8, b8))
    o = jnp.dot(h8, w9[...], preferred_element_type=jnp.float32)
    out_ref[0] = o.T[:13]


def _full(shape):
    nd = len(shape)
    return pl.BlockSpec(shape, lambda b, n: (0,) * nd)


def kernel(x, params):
    p = params
    B, C, N = x.shape
    xt = jnp.swapaxes(x, 1, 2)
    xt_pad = jnp.pad(xt, ((0, 0), (0, 0), (0, _CP - C)))
    x_pad = jnp.pad(x, ((0, 0), (0, _CP - C), (0, 0)))

    idx = pl.pallas_call(
        _knn_body,
        grid=(B, N // _RBLK),
        in_specs=[pl.BlockSpec((1, _RBLK, _CP), lambda b, n: (b, n, 0)),
                  pl.BlockSpec((1, _CP, N), lambda b, n: (b, 0, 0))],
        out_specs=pl.BlockSpec((1, _K, _RBLK), lambda b, n: (b, 0, n)),
        out_shape=jax.ShapeDtypeStruct((B, _K, N), jnp.int32),
    )(xt_pad, x_pad)

    idx_perm = idx.reshape(_NW, _NCHUNK, _CHUNK)
    table = xt_pad.reshape(B * N, _CP)
    feat = table[idx_perm.reshape(-1)]
    feat4 = feat.reshape(B, _K, N, _CP)

    def w32t(w):
        return jnp.concatenate([
            jnp.pad(w[:, :C].T, ((0, _CP - C), (0, 0))),
            jnp.pad(w[:, C:].T, ((0, _CP - C), (0, 0))),
        ], axis=0)

    def v2(a):
        return a.reshape(1, -1)

    k3_ws = [
        w32t(p['W1']), p['W1_1'].T, p['W2'].T,
        v2(p['bn1_g']), v2(p['bn1_b']), v2(p['bn1_1_g']), v2(p['bn1_1_b']),
        v2(p['bn2_g']), v2(p['bn2_b']),
        w32t(p['W3']), p['W3_1'].T, p['W4'].T,
        v2(p['bn3_g']), v2(p['bn3_b']), v2(p['bn3_1_g']), v2(p['bn3_1_b']),
        v2(p['bn4_g']), v2(p['bn4_b']),
        w32t(p['W5']), p['W5_1'].T, p['W5_2'].T,
        v2(p['bn5_g']), v2(p['bn5_b']), v2(p['bn5_1_g']), v2(p['bn5_1_b']),
        v2(p['bn5_2_g']), v2(p['bn5_2_b']),
        p['W6'].T, v2(p['bn6_g']), v2(p['bn6_b']),
    ]

    x123, gmax = pl.pallas_call(
        _k3_body,
        grid=(B, N // _P),
        in_specs=[pl.BlockSpec((1, _K, _P, _CP), lambda b, n: (b, 0, n, 0)),
                  pl.BlockSpec((1, _P, _CP), lambda b, n: (b, n, 0))]
                 + [_full(w.shape) for w in k3_ws],
        out_specs=[pl.BlockSpec((1, _P, 192), lambda b, n: (b, n, 0)),
                   pl.BlockSpec((1, 1, 1024), lambda b, n: (b, 0, 0))],
        out_shape=[jax.ShapeDtypeStruct((B, N, 192), jnp.float32),
                   jax.ShapeDtypeStruct((B, 1, 1024), jnp.float32)],
    )(feat4, xt_pad, *k3_ws)

    k4_ws = [
        p['W7'][:, :1024].T, p['W7'][:, 1024:].T,
        v2(p['bn7_g']), v2(p['bn7_b']),
        p['W8'].T, v2(p['bn8_g']), v2(p['bn8_b']),
        jnp.pad(p['W9'], ((0, 3), (0, 0))).T,
    ]

    out = pl.pallas_call(
        _k4_body,
        grid=(B, N // _P),
        in_specs=[pl.BlockSpec((1, _P, 192), lambda b, n: (b, n, 0)),
                  pl.BlockSpec((1, 1, 1024), lambda b, n: (b, 0, 0))]
                 + [_full(w.shape) for w in k4_ws],
        out_specs=pl.BlockSpec((1, 13, _P), lambda b, n: (b, 0, n)),
        out_shape=jax.ShapeDtypeStruct((B, 13, N), jnp.float32),
    )(x123, gmax, *k4_ws)

    return out


# speedup vs baseline: 1.6640x; 1.6640x over previous
import functools
import numpy as np
import jax
import jax.numpy as jnp
from jax import lax
from jax.experimental import pallas as pl
from jax.experimental.pallas import tpu as pltpu
from jax.experimental.pallas import tpu_sc as plsc

_B, _N, _K, _CP = 4, 4096, 20, 16
_RBLK = 256
_P = 256
_SQ = float(np.sqrt(np.float32(1.0 + 1e-5)))

_NC, _NS = 2, 16
_NW = _NC * _NS
_TOT = _B * _K * _N
_PW = _TOT // _NW
_CHUNK = 128
_NCHUNK = _PW // _CHUNK
_GCH = 16
_GR = _GCH * _CHUNK
_NGRP = _NCHUNK // _GCH


def _knn_body(xt_ref, xa_ref, idx_ref):
    b = pl.program_id(0)
    xt = xt_ref[0]
    xa = xa_ref[0]
    g = jnp.dot(xt, xa, preferred_element_type=jnp.float32)
    sqr = xt * xt
    sqc = xa * xa
    rn = sqr[:, 0:1]
    cn = sqc[0:1, :]
    for c in range(1, 9):
        rn = rn + sqr[:, c:c + 1]
        cn = cn + sqc[c:c + 1, :]
    d = (2.0 * g - cn) - rn
    fiota = -lax.broadcasted_iota(jnp.int32, d.shape, 1).astype(jnp.float32)
    neg = jnp.float32(-jnp.inf)
    negbig = jnp.float32(-3.0e38)
    cols = []
    for _ in range(_K):
        m = jnp.max(d, axis=1, keepdims=True)
        sel = jnp.where(d == m, fiota, negbig)
        mx = jnp.max(sel, axis=1, keepdims=True)
        cols.append(mx)
        d = jnp.where(sel == mx, neg, d)
    idxf = jnp.concatenate(cols, axis=1)
    idx_ref[0] = b * _N - idxf.T.astype(jnp.int32)


def _sc_gather_body(table_ref, idx_ref, out_ref, idx_v, rows_v, sem):
    wid = lax.axis_index("s") * _NC + lax.axis_index("c")
    pltpu.sync_copy(idx_ref.at[wid], idx_v)

    def group(gi, carry):
        cps = [
            pltpu.async_copy(
                table_ref.at[idx_v.at[gi * _GCH + j]],
                rows_v.at[pl.ds(j * _CHUNK, _CHUNK)],
                sem,
            )
            for j in range(_GCH)
        ]
        for cp in cps:
            cp.wait()
        pltpu.sync_copy(rows_v, out_ref.at[pl.ds(wid * _PW + gi * _GR, _GR)])
        return carry

    lax.fori_loop(0, _NGRP, group, 0)


def _sc_gather(table, idx3):
    mesh = plsc.VectorSubcoreMesh(core_axis_name="c", subcore_axis_name="s")
    f = pl.kernel(
        _sc_gather_body,
        mesh=mesh,
        out_type=jax.ShapeDtypeStruct((_TOT, _CP), jnp.float32),
        scratch_types=[
            pltpu.VMEM((_NCHUNK, _CHUNK), jnp.int32),
            pltpu.VMEM((_GR, _CP), jnp.float32),
            pltpu.SemaphoreType.DMA,
        ],
        compiler_params=pltpu.CompilerParams(use_tc_tiling_on_sc=False),
    )
    return f(table, idx3)


def _lrelu(h):
    return jnp.where(h >= 0, h, 0.2 * h)


def _bn(h, g, b):
    return h * (g[...] / _SQ) + b[...]


def _k3_body(feat_ref, xe_ref,
             w1t, w11, w2, g1, b1, g11, b11, g2, b2,
             w3t, w31, w4, g3, b3, g31, b31, g4, b4,
             w5t, w51, w52, g5, b5, g51, b51, g52, b52,
             w6, g6, b6,
             x123_ref, gmax_ref):
    nb = pl.program_id(1)
    f20 = feat_ref[0].reshape(_K * _P, _CP)
    xe = xe_ref[0]

    def branch(k, wa, wb, wc, ga, ba, gb, bb, gc, bc):
        f = f20[: k * _P]
        xe_t = jnp.concatenate([xe] * k, axis=0)
        e1in = jnp.concatenate([f - xe_t, xe_t], axis=1)
        e1 = _lrelu(_bn(jnp.dot(e1in, wa[...], preferred_element_type=jnp.float32), ga, ba))
        e2 = _lrelu(_bn(jnp.dot(e1, wb[...], preferred_element_type=jnp.float32), gb, bb))
        e3 = _lrelu(_bn(jnp.dot(e2, wc[...], preferred_element_type=jnp.float32), gc, bc))
        m = e3[:_P]
        for j in range(1, k):
            m = jnp.maximum(m, e3[j * _P:(j + 1) * _P])
        return m

    x1 = branch(20, w1t, w11, w2, g1, b1, g11, b11, g2, b2)
    x2 = branch(6, w3t, w31, w4, g3, b3, g31, b31, g4, b4)
    x3 = branch(8, w5t, w51, w52, g5, b5, g51, b51, g52, b52)
    x123 = jnp.concatenate([x1, x2, x3], axis=1)
    x123_ref[0] = x123
    h6 = _lrelu(_bn(jnp.dot(x123, w6[...], preferred_element_type=jnp.float32), g6, b6))
    bm = jnp.max(h6, axis=0, keepdims=True)

    @pl.when(nb == 0)
    def _init():
        gmax_ref[0] = bm

    @pl.when(nb > 0)
    def _acc():
        gmax_ref[0] = jnp.maximum(gmax_ref[0], bm)


def _k4_body(x123_ref, g_ref, w7g, w7x, g7, b7, w8, g8, b8, w9, out_ref):
    x123 = x123_ref[0]
    gv = g_ref[0]
    c = jnp.dot(gv, w7g[...], preferred_element_type=jnp.float32)
    h7 = _lrelu(_bn(jnp.dot(x123, w7x[...], preferred_element_type=jnp.float32) + c, g7, b7))
    h8 = _lrelu(_bn(jnp.dot(h7, w8[...], preferred_element_type=jnp.float32), g8, b8))
    o = jnp.dot(h8, w9[...], preferred_element_type=jnp.float32)
    out_ref[0] = o.T[:13]


def _full(shape):
    nd = len(shape)
    return pl.BlockSpec(shape, lambda b, n: (0,) * nd)


def kernel(x, params):
    p = params
    B, C, N = x.shape
    xt = jnp.swapaxes(x, 1, 2)
    xt_pad = jnp.pad(xt, ((0, 0), (0, 0), (0, _CP - C)))
    x_pad = jnp.pad(x, ((0, 0), (0, _CP - C), (0, 0)))

    idx = pl.pallas_call(
        _knn_body,
        grid=(B, N // _RBLK),
        in_specs=[pl.BlockSpec((1, _RBLK, _CP), lambda b, n: (b, n, 0)),
                  pl.BlockSpec((1, _CP, N), lambda b, n: (b, 0, 0))],
        out_specs=pl.BlockSpec((1, _K, _RBLK), lambda b, n: (b, 0, n)),
        out_shape=jax.ShapeDtypeStruct((B, _K, N), jnp.int32),
    )(xt_pad, x_pad)

    idx_perm = idx.reshape(_NW, _NCHUNK, _CHUNK)
    table = xt_pad.reshape(B * N, _CP)
    feat = _sc_gather(table, idx_perm)
    feat4 = feat.reshape(B, _K, N, _CP)

    def w32t(w):
        return jnp.concatenate([
            jnp.pad(w[:, :C].T, ((0, _CP - C), (0, 0))),
            jnp.pad(w[:, C:].T, ((0, _CP - C), (0, 0))),
        ], axis=0)

    def v2(a):
        return a.reshape(1, -1)

    k3_ws = [
        w32t(p['W1']), p['W1_1'].T, p['W2'].T,
        v2(p['bn1_g']), v2(p['bn1_b']), v2(p['bn1_1_g']), v2(p['bn1_1_b']),
        v2(p['bn2_g']), v2(p['bn2_b']),
        w32t(p['W3']), p['W3_1'].T, p['W4'].T,
        v2(p['bn3_g']), v2(p['bn3_b']), v2(p['bn3_1_g']), v2(p['bn3_1_b']),
        v2(p['bn4_g']), v2(p['bn4_b']),
        w32t(p['W5']), p['W5_1'].T, p['W5_2'].T,
        v2(p['bn5_g']), v2(p['bn5_b']), v2(p['bn5_1_g']), v2(p['bn5_1_b']),
        v2(p['bn5_2_g']), v2(p['bn5_2_b']),
        p['W6'].T, v2(p['bn6_g']), v2(p['bn6_b']),
    ]

    x123, gmax = pl.pallas_call(
        _k3_body,
        grid=(B, N // _P),
        in_specs=[pl.BlockSpec((1, _K, _P, _CP), lambda b, n: (b, 0, n, 0)),
                  pl.BlockSpec((1, _P, _CP), lambda b, n: (b, n, 0))]
                 + [_full(w.shape) for w in k3_ws],
        out_specs=[pl.BlockSpec((1, _P, 192), lambda b, n: (b, n, 0)),
                   pl.BlockSpec((1, 1, 1024), lambda b, n: (b, 0, 0))],
        out_shape=[jax.ShapeDtypeStruct((B, N, 192), jnp.float32),
                   jax.ShapeDtypeStruct((B, 1, 1024), jnp.float32)],
    )(feat4, xt_pad, *k3_ws)

    k4_ws = [
        p['W7'][:, :1024].T, p['W7'][:, 1024:].T,
        v2(p['bn7_g']), v2(p['bn7_b']),
        p['W8'].T, v2(p['bn8_g']), v2(p['bn8_b']),
        jnp.pad(p['W9'], ((0, 3), (0, 0))).T,
    ]

    out = pl.pallas_call(
        _k4_body,
        grid=(B, N // _P),
        in_specs=[pl.BlockSpec((1, _P, 192), lambda b, n: (b, n, 0)),
                  pl.BlockSpec((1, 1, 1024), lambda b, n: (b, 0, 0))]
                 + [_full(w.shape) for w in k4_ws],
        out_specs=pl.BlockSpec((1, 13, _P), lambda b, n: (b, 0, n)),
        out_shape=jax.ShapeDtypeStruct((B, 13, N), jnp.float32),
    )(x123, gmax, *k4_ws)

    return out
